# baseline (speedup 1.0000x reference)
import functools

import jax
import jax.numpy as jnp
from jax import lax
from jax.experimental import pallas as pl
from jax.experimental.pallas import tpu as pltpu
from jax.experimental.pallas import tpu_sc as plsc

B = 2
D_MODEL = 256
N_HEADS = 8
N_LEVELS = 4
N_POINTS = 4
D_PH = D_MODEL // N_HEADS
LEVEL_SHAPES = ((64, 64), (32, 32), (16, 16), (8, 8))
STARTS = (0, 4096, 5120, 5376)
LIN = 5440
LQ = 5440
NCOL = N_HEADS * N_LEVELS * N_POINTS
NTERM = 4 * NCOL

LB = 544

import numpy as _np
_CH_PERM = _np.concatenate([_np.arange(0, 32, 2), _np.arange(1, 32, 2)])


def _prep_body(q_ref, inf_ref, rp_ref, wv_ref, bv_ref, woff_ref, boff_ref,
               wa_ref, ba_ref, val_ref, idx_ref, w_ref):
    b = pl.program_id(0)
    q = q_ref[0]
    inf = inf_ref[0]
    f32 = jnp.float32
    val_ref[0] = (jnp.dot(inf, wv_ref[...], preferred_element_type=f32) + bv_ref[...]).astype(jnp.bfloat16)

    offxy = jnp.dot(q, woff_ref[...], preferred_element_type=f32) + boff_ref[...]
    offx = offxy[:, :NCOL]
    offy = offxy[:, NCOL:]

    logits = jnp.dot(q, wa_ref[...], preferred_element_type=f32) + ba_ref[...]
    lg = logits.reshape(LB, N_HEADS, N_LEVELS * N_POINTS)
    m = jnp.max(lg, axis=-1, keepdims=True)
    e = jnp.exp(lg - m)
    aw = (e / jnp.sum(e, axis=-1, keepdims=True)).reshape(LB, NCOL)

    c1 = lax.broadcasted_iota(jnp.int32, (1, NCOL), 1)
    lc = (c1 // N_POINTS) % N_LEVELS
    hc = c1 // (N_LEVELS * N_POINTS)
    wci = 64 >> lc
    wcf = wci.astype(f32)
    sc = jnp.where(lc == 0, 0, jnp.where(lc == 1, 4096, jnp.where(lc == 2, 5120, 5376)))

    rows8 = lax.broadcasted_iota(jnp.int32, (8, NCOL), 0)
    lcol8 = (lax.broadcasted_iota(jnp.int32, (8, NCOL), 1) // N_POINTS) % N_LEVELS
    sx = (rows8 == 2 * lcol8).astype(f32)
    sy = (rows8 == 2 * lcol8 + 1).astype(f32)
    rp = rp_ref[0]
    refx = jnp.dot(rp, sx, preferred_element_type=f32, precision=jax.lax.Precision.HIGHEST)
    refy = jnp.dot(rp, sy, preferred_element_type=f32, precision=jax.lax.Precision.HIGHEST)

    ix = refx * wcf + offx - 0.5
    iy = refy * wcf + offy - 0.5
    x0 = jnp.floor(ix)
    y0 = jnp.floor(iy)
    fx = ix - x0
    fy = iy - y0

    idx_parts = []
    w_parts = []
    for corner in range(4):
        dx = corner & 1
        dy = corner >> 1
        cx = x0 + dx
        cy = y0 + dy
        wx = fx if dx else 1.0 - fx
        wy = fy if dy else 1.0 - fy
        valid = (cx >= 0) & (cx <= wcf - 1) & (cy >= 0) & (cy <= wcf - 1)
        wgt = aw * wx * wy * jnp.where(valid, 1.0, 0.0)
        xi = jnp.clip(cx, 0.0, wcf - 1).astype(jnp.int32)
        yi = jnp.clip(cy, 0.0, wcf - 1).astype(jnp.int32)
        idx = (b * LIN + sc + yi * wci + xi) * N_HEADS + hc
        idx_parts.append(idx)
        w_parts.append(wgt)
    idx_ref[0] = jnp.concatenate(idx_parts, axis=1)
    p01 = jnp.stack([w_parts[0], w_parts[1]], axis=-1).reshape(LB, 2 * NCOL)
    p23 = jnp.stack([w_parts[2], w_parts[3]], axis=-1).reshape(LB, 2 * NCOL)
    w_ref[0] = jnp.concatenate([p01, p23], axis=1).astype(jnp.bfloat16)


def _prep(query, input_flatten, rp8, wv, bv2, woff_p, boff_p2, wa, ba2):
    nb = LQ // LB
    grid = (B, nb)
    row_spec = lambda n: pl.BlockSpec((1, LB, n), lambda b, i: (b, i, 0))
    full = lambda shape: pl.BlockSpec(shape, lambda b, i: (0,) * len(shape))
    return pl.pallas_call(
        _prep_body,
        grid=grid,
        in_specs=[
            row_spec(D_MODEL),
            row_spec(D_MODEL),
            row_spec(8),
            full((D_MODEL, D_MODEL)),
            full((1, D_MODEL)),
            full((D_MODEL, 2 * NCOL)),
            full((1, 2 * NCOL)),
            full((D_MODEL, NCOL)),
            full((1, NCOL)),
        ],
        out_specs=[
            row_spec(D_MODEL),
            row_spec(NTERM),
            row_spec(NTERM),
        ],
        out_shape=[
            jax.ShapeDtypeStruct((B, LQ, D_MODEL), jnp.bfloat16),
            jax.ShapeDtypeStruct((B, LQ, NTERM), jnp.int32),
            jax.ShapeDtypeStruct((B, LQ, NTERM), jnp.bfloat16),
        ],
    )(query, input_flatten, rp8, wv, bv2, woff_p, boff_p2, wa, ba2)


NC = 2
NS = 16
NW = NC * NS
GROUPS = B * LQ
PER_W = GROUPS // NW
CH = 5
NCHUNK = PER_W // CH


def _sc_body(table_hbm, idx_hbm, w_hbm, out_hbm, idx_v, w_v, data_v, out_v,
             gsem, osem, iosem):
    wid = lax.axis_index("s") * NC + lax.axis_index("c")
    base = wid * PER_W

    def issue_idx(ci, s):
        g0 = base + ci * CH
        pltpu.async_copy(idx_hbm.at[pl.ds(g0 * NTERM, CH * NTERM)], idx_v.at[s],
                         iosem)

    def issue_w(ci, s):
        g0 = base + ci * CH
        pltpu.async_copy(w_hbm.at[pl.ds(g0, CH)], w_v.at[s], iosem)

    def wait_io(s):
        pltpu.make_async_copy(idx_hbm.at[pl.ds(0, CH * NTERM)], idx_v.at[s],
                              iosem).wait()
        pltpu.make_async_copy(w_hbm.at[pl.ds(0, CH)], w_v.at[s], iosem).wait()

    def issue_gathers(s):
        pltpu.async_copy(table_hbm.at[idx_v.at[s]], data_v.at[s], gsem)

    def wait_gathers(s):
        pltpu.make_async_copy(table_hbm.at[idx_v.at[s]], data_v.at[s],
                              gsem).wait()

    def drain_out(s):
        pltpu.make_async_copy(out_v.at[s], out_hbm.at[pl.ds(0, CH * N_HEADS)],
                              osem).wait()

    pltpu.sync_copy(idx_hbm.at[pl.ds(base * NTERM, CH * NTERM)], idx_v.at[0])
    pltpu.sync_copy(w_hbm.at[pl.ds(base, CH)], w_v.at[0])
    issue_gathers(0)
    issue_idx(1, 1)
    issue_w(1, 1)

    def chunk_body(ci, carry):
        s = lax.rem(ci, 2)
        g0 = base + ci * CH

        @pl.when(ci + 1 < NCHUNK)
        def _():
            wait_io(1 - s)
            issue_gathers(1 - s)

        wait_gathers(s)

        @pl.when(ci + 2 < NCHUNK)
        def _():
            issue_idx(ci + 2, s)

        @pl.when(ci >= 2)
        def _():
            drain_out(s)

        def g_h_body(gh, carry2):
            g = gh // N_HEADS
            h = gh % N_HEADS
            acc0 = jnp.zeros((16,), jnp.float32)
            acc1 = jnp.zeros((16,), jnp.float32)
            for cpair in range(2):
                wv32 = w_v[s, g, pl.ds(cpair * 2 * NCOL + h * 32, 32)]
                wa16, wb16 = plsc.unpack(wv32,
                                         format=plsc.PackFormat.INTERLEAVED)
                for k in range(16):
                    wa_s = wa16[k]
                    wb_s = wb16[k]
                    rowa = g * NTERM + (cpair * 2) * NCOL + h * 16 + k
                    rowb = g * NTERM + (cpair * 2 + 1) * NCOL + h * 16 + k
                    a0, a1 = plsc.unpack(data_v[s, rowa, :],
                                         format=plsc.PackFormat.INTERLEAVED)
                    b0, b1 = plsc.unpack(data_v[s, rowb, :],
                                         format=plsc.PackFormat.INTERLEAVED)
                    acc0 = acc0 + wa_s * a0 + wb_s * b0
                    acc1 = acc1 + wa_s * a1 + wb_s * b1
            out_v[s, gh, pl.ds(0, 16)] = acc0
            out_v[s, gh, pl.ds(16, 16)] = acc1
            return carry2

        lax.fori_loop(0, CH * N_HEADS, g_h_body, 0)
        pltpu.async_copy(out_v.at[s],
                         out_hbm.at[pl.ds(g0 * N_HEADS, CH * N_HEADS)], osem)

        @pl.when(ci + 2 < NCHUNK)
        def _():
            issue_w(ci + 2, s)

        return carry

    lax.fori_loop(0, NCHUNK, chunk_body, 0)
    drain_out(0)
    drain_out(1)


def _sc_gather(table, idx_flat, w2):
    mesh = plsc.VectorSubcoreMesh(core_axis_name="c", subcore_axis_name="s")
    k = pl.kernel(
        _sc_body,
        out_type=jax.ShapeDtypeStruct((GROUPS * N_HEADS, D_PH), jnp.float32),
        mesh=mesh,
        scratch_types=[
            pltpu.VMEM((2, CH * NTERM), jnp.int32),
            pltpu.VMEM((2, CH, NTERM), jnp.bfloat16),
            pltpu.VMEM((2, CH * NTERM, D_PH), jnp.bfloat16),
            pltpu.VMEM((2, CH * N_HEADS, D_PH), jnp.float32),
            pltpu.SemaphoreType.DMA,
            pltpu.SemaphoreType.DMA,
            pltpu.SemaphoreType.DMA,
        ],
        compiler_params=pltpu.CompilerParams(use_tc_tiling_on_sc=False,
                                             needs_layout_passes=False),
    )
    return k(table, idx_flat, w2)


def _proj_body(x_ref, w_ref, b_ref, o_ref):
    o_ref[...] = (jnp.dot(x_ref[...], w_ref[...],
                          preferred_element_type=jnp.float32) + b_ref[...])


def _out_proj(x, wout, bout2):
    nb = (B * LQ) // LB
    return pl.pallas_call(
        _proj_body,
        grid=(nb,),
        in_specs=[
            pl.BlockSpec((LB, D_MODEL), lambda i: (i, 0)),
            pl.BlockSpec((D_MODEL, D_MODEL), lambda i: (0, 0)),
            pl.BlockSpec((1, D_MODEL), lambda i: (0, 0)),
        ],
        out_specs=pl.BlockSpec((LB, D_MODEL), lambda i: (i, 0)),
        out_shape=jax.ShapeDtypeStruct((B * LQ, D_MODEL), jnp.float32),
    )(x, wout, bout2)


def kernel(query, reference_points, input_flatten, input_spatial_shapes,
           input_level_start_index, Wv, bv, Woff, boff, Wa, ba, Wout, bout):
    del input_spatial_shapes, input_level_start_index
    rp8 = reference_points.reshape(B, LQ, 8)
    woff_p = Woff.reshape(D_MODEL, NCOL, 2).transpose(0, 2, 1).reshape(D_MODEL, 2 * NCOL)
    boff_p = boff.reshape(NCOL, 2).transpose(1, 0).reshape(1, 2 * NCOL)

    value, idx, w = _prep(query, input_flatten, rp8, Wv, bv.reshape(1, -1),
                          woff_p, boff_p, Wa, ba.reshape(1, -1))

    table = value.reshape(B * LIN * N_HEADS, D_PH)
    idx_flat = idx.reshape(GROUPS * NTERM)
    w2 = w.reshape(GROUPS, NTERM)
    sampled = _sc_gather(table, idx_flat, w2)

    ar = jnp.arange(D_MODEL)
    perm = (ar // D_PH) * D_PH + jnp.asarray(_CH_PERM)[ar % D_PH]
    out = _out_proj(sampled.reshape(B * LQ, D_MODEL), Wout[perm], bout.reshape(1, -1))
    return out.reshape(B, LQ, D_MODEL)


# speedup vs baseline: 3.5592x; 3.5592x over previous
import functools

import jax
import jax.numpy as jnp
from jax import lax
from jax.experimental import pallas as pl
from jax.experimental.pallas import tpu as pltpu
from jax.experimental.pallas import tpu_sc as plsc

B = 2
D_MODEL = 256
N_HEADS = 8
N_LEVELS = 4
N_POINTS = 4
D_PH = D_MODEL // N_HEADS
LEVEL_SHAPES = ((64, 64), (32, 32), (16, 16), (8, 8))
STARTS = (0, 4096, 5120, 5376)
LIN = 5440
LQ = 5440
NCOL = N_HEADS * N_LEVELS * N_POINTS
NTERM = 4 * NCOL

LB = 544

import numpy as _np
_CH_PERM = _np.concatenate([_np.arange(0, 32, 2), _np.arange(1, 32, 2)])


def _prep_body(q_ref, inf_ref, rp_ref, wv_ref, bv_ref, woff_ref, boff_ref,
               wa_ref, ba_ref, val_ref, idx_ref, w_ref):
    b = pl.program_id(0)
    q = q_ref[0]
    inf = inf_ref[0]
    f32 = jnp.float32
    val_ref[0] = (jnp.dot(inf, wv_ref[...], preferred_element_type=f32) + bv_ref[...]).astype(jnp.bfloat16)

    offxy = jnp.dot(q, woff_ref[...], preferred_element_type=f32) + boff_ref[...]
    offx = offxy[:, :NCOL]
    offy = offxy[:, NCOL:]

    logits = jnp.dot(q, wa_ref[...], preferred_element_type=f32) + ba_ref[...]
    lg = logits.reshape(LB, N_HEADS, N_LEVELS * N_POINTS)
    m = jnp.max(lg, axis=-1, keepdims=True)
    e = jnp.exp(lg - m)
    aw = (e / jnp.sum(e, axis=-1, keepdims=True)).reshape(LB, NCOL)

    c1 = lax.broadcasted_iota(jnp.int32, (1, NCOL), 1)
    lc = (c1 // N_POINTS) % N_LEVELS
    hc = c1 // (N_LEVELS * N_POINTS)
    wci = 64 >> lc
    wcf = wci.astype(f32)
    sc = jnp.where(lc == 0, 0, jnp.where(lc == 1, 4096, jnp.where(lc == 2, 5120, 5376)))

    rows8 = lax.broadcasted_iota(jnp.int32, (8, NCOL), 0)
    lcol8 = (lax.broadcasted_iota(jnp.int32, (8, NCOL), 1) // N_POINTS) % N_LEVELS
    sx = (rows8 == 2 * lcol8).astype(f32)
    sy = (rows8 == 2 * lcol8 + 1).astype(f32)
    rp = rp_ref[0]
    refx = jnp.dot(rp, sx, preferred_element_type=f32, precision=jax.lax.Precision.HIGHEST)
    refy = jnp.dot(rp, sy, preferred_element_type=f32, precision=jax.lax.Precision.HIGHEST)

    ix = refx * wcf + offx - 0.5
    iy = refy * wcf + offy - 0.5
    x0 = jnp.floor(ix)
    y0 = jnp.floor(iy)
    fx = ix - x0
    fy = iy - y0

    idx_parts = []
    w_parts = []
    for corner in range(4):
        dx = corner & 1
        dy = corner >> 1
        cx = x0 + dx
        cy = y0 + dy
        wx = fx if dx else 1.0 - fx
        wy = fy if dy else 1.0 - fy
        valid = (cx >= 0) & (cx <= wcf - 1) & (cy >= 0) & (cy <= wcf - 1)
        wgt = aw * wx * wy * jnp.where(valid, 1.0, 0.0)
        xi = jnp.clip(cx, 0.0, wcf - 1).astype(jnp.int32)
        yi = jnp.clip(cy, 0.0, wcf - 1).astype(jnp.int32)
        idx = (b * LIN + sc + yi * wci + xi) * N_HEADS + hc
        idx_parts.append(idx)
        w_parts.append(wgt)
    idx_ref[0] = jnp.concatenate(idx_parts, axis=1)
    w_ref[0] = jnp.concatenate(w_parts, axis=1)


def _prep(query, input_flatten, rp8, wv, bv2, woff_p, boff_p2, wa, ba2):
    nb = LQ // LB
    grid = (B, nb)
    row_spec = lambda n: pl.BlockSpec((1, LB, n), lambda b, i: (b, i, 0))
    full = lambda shape: pl.BlockSpec(shape, lambda b, i: (0,) * len(shape))
    return pl.pallas_call(
        _prep_body,
        grid=grid,
        in_specs=[
            row_spec(D_MODEL),
            row_spec(D_MODEL),
            row_spec(8),
            full((D_MODEL, D_MODEL)),
            full((1, D_MODEL)),
            full((D_MODEL, 2 * NCOL)),
            full((1, 2 * NCOL)),
            full((D_MODEL, NCOL)),
            full((1, NCOL)),
        ],
        out_specs=[
            row_spec(D_MODEL),
            row_spec(NTERM),
            row_spec(NTERM),
        ],
        out_shape=[
            jax.ShapeDtypeStruct((B, LQ, D_MODEL), jnp.bfloat16),
            jax.ShapeDtypeStruct((B, LQ, NTERM), jnp.int32),
            jax.ShapeDtypeStruct((B, LQ, NTERM), jnp.float32),
        ],
    )(query, input_flatten, rp8, wv, bv2, woff_p, boff_p2, wa, ba2)


NC = 2
NS = 16
NW = NC * NS
GROUPS = B * LQ
PER_W = GROUPS // NW
CH = 5
NCHUNK = PER_W // CH


def _sc_body(table_hbm, idx_hbm, w_hbm, out_hbm, idx_v, w_v, data_v, out_v,
             gsem, osem, iosem):
    wid = lax.axis_index("s") * NC + lax.axis_index("c")
    base = wid * PER_W

    def issue_idx(ci, s):
        g0 = base + ci * CH
        pltpu.async_copy(idx_hbm.at[pl.ds(g0 * NTERM, CH * NTERM)], idx_v.at[s],
                         iosem)

    def issue_w(ci, s):
        g0 = base + ci * CH
        pltpu.async_copy(w_hbm.at[pl.ds(g0, CH)], w_v.at[s], iosem)

    def wait_io(s):
        pltpu.make_async_copy(idx_hbm.at[pl.ds(0, CH * NTERM)], idx_v.at[s],
                              iosem).wait()
        pltpu.make_async_copy(w_hbm.at[pl.ds(0, CH)], w_v.at[s], iosem).wait()

    def issue_gathers(s):
        pltpu.async_copy(table_hbm.at[idx_v.at[s]], data_v.at[s], gsem)

    def wait_gathers(s):
        pltpu.make_async_copy(table_hbm.at[idx_v.at[s]], data_v.at[s],
                              gsem).wait()

    def drain_out(s):
        pltpu.make_async_copy(out_v.at[s], out_hbm.at[pl.ds(0, CH * N_HEADS)],
                              osem).wait()

    pltpu.sync_copy(idx_hbm.at[pl.ds(base * NTERM, CH * NTERM)], idx_v.at[0])
    pltpu.sync_copy(w_hbm.at[pl.ds(base, CH)], w_v.at[0])
    issue_gathers(0)
    issue_idx(1, 1)
    issue_w(1, 1)

    def chunk_body(ci, carry):
        s = lax.rem(ci, 2)
        g0 = base + ci * CH

        @pl.when(ci + 1 < NCHUNK)
        def _():
            wait_io(1 - s)
            issue_gathers(1 - s)

        wait_gathers(s)

        @pl.when(ci + 2 < NCHUNK)
        def _():
            issue_idx(ci + 2, s)

        @pl.when(ci >= 2)
        def _():
            drain_out(s)

        def g_h_body(gh, carry2):
            g = gh // N_HEADS
            h = gh % N_HEADS
            acc0 = jnp.zeros((16,), jnp.float32)
            acc1 = jnp.zeros((16,), jnp.float32)
            for corner in range(4):
                w16 = w_v[s, g, pl.ds(corner * NCOL + h * 16, 16)]
                for k in range(16):
                    wv_s = w16[k]
                    row = g * NTERM + corner * NCOL + h * 16 + k
                    d0, d1 = plsc.unpack(data_v[s, row, :],
                                         format=plsc.PackFormat.INTERLEAVED)
                    acc0 = acc0 + wv_s * d0
                    acc1 = acc1 + wv_s * d1
            out_v[s, gh, pl.ds(0, 16)] = acc0
            out_v[s, gh, pl.ds(16, 16)] = acc1
            return carry2

        lax.fori_loop(0, CH * N_HEADS, g_h_body, 0)
        pltpu.async_copy(out_v.at[s],
                         out_hbm.at[pl.ds(g0 * N_HEADS, CH * N_HEADS)], osem)

        @pl.when(ci + 2 < NCHUNK)
        def _():
            issue_w(ci + 2, s)

        return carry

    lax.fori_loop(0, NCHUNK, chunk_body, 0)
    drain_out(0)
    drain_out(1)


def _sc_gather(table, idx_flat, w2):
    mesh = plsc.VectorSubcoreMesh(core_axis_name="c", subcore_axis_name="s")
    k = pl.kernel(
        _sc_body,
        out_type=jax.ShapeDtypeStruct((GROUPS * N_HEADS, D_PH), jnp.float32),
        mesh=mesh,
        scratch_types=[
            pltpu.VMEM((2, CH * NTERM), jnp.int32),
            pltpu.VMEM((2, CH, NTERM), jnp.float32),
            pltpu.VMEM((2, CH * NTERM, D_PH), jnp.bfloat16),
            pltpu.VMEM((2, CH * N_HEADS, D_PH), jnp.float32),
            pltpu.SemaphoreType.DMA,
            pltpu.SemaphoreType.DMA,
            pltpu.SemaphoreType.DMA,
        ],
        compiler_params=pltpu.CompilerParams(use_tc_tiling_on_sc=False,
                                             needs_layout_passes=False),
    )
    return k(table, idx_flat, w2)


def _proj_body(x_ref, w_ref, b_ref, o_ref):
    o_ref[...] = (jnp.dot(x_ref[...], w_ref[...],
                          preferred_element_type=jnp.float32) + b_ref[...])


def _out_proj(x, wout, bout2):
    nb = (B * LQ) // LB
    return pl.pallas_call(
        _proj_body,
        grid=(nb,),
        in_specs=[
            pl.BlockSpec((LB, D_MODEL), lambda i: (i, 0)),
            pl.BlockSpec((D_MODEL, D_MODEL), lambda i: (0, 0)),
            pl.BlockSpec((1, D_MODEL), lambda i: (0, 0)),
        ],
        out_specs=pl.BlockSpec((LB, D_MODEL), lambda i: (i, 0)),
        out_shape=jax.ShapeDtypeStruct((B * LQ, D_MODEL), jnp.float32),
    )(x, wout, bout2)


def kernel(query, reference_points, input_flatten, input_spatial_shapes,
           input_level_start_index, Wv, bv, Woff, boff, Wa, ba, Wout, bout):
    del input_spatial_shapes, input_level_start_index
    rp8 = reference_points.reshape(B, LQ, 8)
    woff_p = Woff.reshape(D_MODEL, NCOL, 2).transpose(0, 2, 1).reshape(D_MODEL, 2 * NCOL)
    boff_p = boff.reshape(NCOL, 2).transpose(1, 0).reshape(1, 2 * NCOL)

    value, idx, w = _prep(query, input_flatten, rp8, Wv, bv.reshape(1, -1),
                          woff_p, boff_p, Wa, ba.reshape(1, -1))

    table = value.reshape(B * LIN * N_HEADS, D_PH)
    idx_flat = idx.reshape(GROUPS * NTERM)
    w2 = w.reshape(GROUPS, NTERM)
    sampled = _sc_gather(table, idx_flat, w2)

    ar = jnp.arange(D_MODEL)
    perm = (ar // D_PH) * D_PH + jnp.asarray(_CH_PERM)[ar % D_PH]
    out = _out_proj(sampled.reshape(B * LQ, D_MODEL), Wout[perm], bout.reshape(1, -1))
    return out.reshape(B, LQ, D_MODEL)


# speedup vs baseline: 3.5925x; 1.0094x over previous
import functools

import jax
import jax.numpy as jnp
from jax import lax
from jax.experimental import pallas as pl
from jax.experimental.pallas import tpu as pltpu
from jax.experimental.pallas import tpu_sc as plsc

B = 2
D_MODEL = 256
N_HEADS = 8
N_LEVELS = 4
N_POINTS = 4
D_PH = D_MODEL // N_HEADS
LEVEL_SHAPES = ((64, 64), (32, 32), (16, 16), (8, 8))
STARTS = (0, 4096, 5120, 5376)
LIN = 5440
LQ = 5440
NCOL = N_HEADS * N_LEVELS * N_POINTS
NTERM = 4 * NCOL

LB = 544

import numpy as _np
_CH_PERM = _np.concatenate([_np.arange(0, 32, 2), _np.arange(1, 32, 2)])


def _prep_body(q_ref, inf_ref, rp_ref, wv_ref, bv_ref, woff_ref, boff_ref,
               wa_ref, ba_ref, val_ref, idx_ref, w_ref):
    b = pl.program_id(0)
    q = q_ref[0]
    inf = inf_ref[0]
    f32 = jnp.float32
    val_ref[0] = (jnp.dot(inf, wv_ref[...], preferred_element_type=f32) + bv_ref[...]).astype(jnp.bfloat16)

    offxy = jnp.dot(q, woff_ref[...], preferred_element_type=f32) + boff_ref[...]
    offx = offxy[:, :NCOL]
    offy = offxy[:, NCOL:]

    logits = jnp.dot(q, wa_ref[...], preferred_element_type=f32) + ba_ref[...]
    lg = logits.reshape(LB, N_HEADS, N_LEVELS * N_POINTS)
    m = jnp.max(lg, axis=-1, keepdims=True)
    e = jnp.exp(lg - m)
    aw = (e / jnp.sum(e, axis=-1, keepdims=True)).reshape(LB, NCOL)

    c1 = lax.broadcasted_iota(jnp.int32, (1, NCOL), 1)
    lc = (c1 // N_POINTS) % N_LEVELS
    hc = c1 // (N_LEVELS * N_POINTS)
    wci = 64 >> lc
    wcf = wci.astype(f32)
    sc = jnp.where(lc == 0, 0, jnp.where(lc == 1, 4096, jnp.where(lc == 2, 5120, 5376)))

    rows8 = lax.broadcasted_iota(jnp.int32, (8, NCOL), 0)
    lcol8 = (lax.broadcasted_iota(jnp.int32, (8, NCOL), 1) // N_POINTS) % N_LEVELS
    sx = (rows8 == 2 * lcol8).astype(f32)
    sy = (rows8 == 2 * lcol8 + 1).astype(f32)
    rp = rp_ref[0]
    refx = jnp.dot(rp, sx, preferred_element_type=f32, precision=jax.lax.Precision.HIGHEST)
    refy = jnp.dot(rp, sy, preferred_element_type=f32, precision=jax.lax.Precision.HIGHEST)

    ix = refx * wcf + offx - 0.5
    iy = refy * wcf + offy - 0.5
    x0 = jnp.floor(ix)
    y0 = jnp.floor(iy)
    fx = ix - x0
    fy = iy - y0

    idx_parts = []
    w_parts = []
    for corner in range(4):
        dx = corner & 1
        dy = corner >> 1
        cx = x0 + dx
        cy = y0 + dy
        wx = fx if dx else 1.0 - fx
        wy = fy if dy else 1.0 - fy
        valid = (cx >= 0) & (cx <= wcf - 1) & (cy >= 0) & (cy <= wcf - 1)
        wgt = aw * wx * wy * jnp.where(valid, 1.0, 0.0)
        xi = jnp.clip(cx, 0.0, wcf - 1).astype(jnp.int32)
        yi = jnp.clip(cy, 0.0, wcf - 1).astype(jnp.int32)
        idx = (b * LIN + sc + yi * wci + xi) * N_HEADS + hc
        idx_parts.append(idx)
        w_parts.append(wgt)
    idx_ref[0] = jnp.concatenate(idx_parts, axis=1)
    w_ref[0] = jnp.concatenate(w_parts, axis=1)


def _prep(query, input_flatten, rp8, wv, bv2, woff_p, boff_p2, wa, ba2):
    nb = LQ // LB
    grid = (B, nb)
    row_spec = lambda n: pl.BlockSpec((1, LB, n), lambda b, i: (b, i, 0))
    full = lambda shape: pl.BlockSpec(shape, lambda b, i: (0,) * len(shape))
    return pl.pallas_call(
        _prep_body,
        grid=grid,
        in_specs=[
            row_spec(D_MODEL),
            row_spec(D_MODEL),
            row_spec(8),
            full((D_MODEL, D_MODEL)),
            full((1, D_MODEL)),
            full((D_MODEL, 2 * NCOL)),
            full((1, 2 * NCOL)),
            full((D_MODEL, NCOL)),
            full((1, NCOL)),
        ],
        out_specs=[
            row_spec(D_MODEL),
            row_spec(NTERM),
            row_spec(NTERM),
        ],
        out_shape=[
            jax.ShapeDtypeStruct((B, LQ, D_MODEL), jnp.bfloat16),
            jax.ShapeDtypeStruct((B, LQ, NTERM), jnp.int32),
            jax.ShapeDtypeStruct((B, LQ, NTERM), jnp.float32),
        ],
    )(query, input_flatten, rp8, wv, bv2, woff_p, boff_p2, wa, ba2)


NC = 2
NS = 16
NW = NC * NS
GROUPS = B * LQ
PER_W = GROUPS // NW
CH = 5
NCHUNK = PER_W // CH


def _sc_body(table_hbm, idx_hbm, w_hbm, out_hbm, idx_v, w_v, data_v, out_v,
             gsem, osem, iosem):
    wid = lax.axis_index("s") * NC + lax.axis_index("c")
    base = wid * PER_W

    def issue_idx(ci, s):
        g0 = base + ci * CH
        pltpu.async_copy(idx_hbm.at[pl.ds(g0 * NTERM, CH * NTERM)], idx_v.at[s],
                         iosem)

    def issue_w(ci, s):
        g0 = base + ci * CH
        pltpu.async_copy(w_hbm.at[pl.ds(g0, CH)], w_v.at[s], iosem)

    def wait_io(s):
        pltpu.make_async_copy(idx_hbm.at[pl.ds(0, CH * NTERM)], idx_v.at[s],
                              iosem).wait()
        pltpu.make_async_copy(w_hbm.at[pl.ds(0, CH)], w_v.at[s], iosem).wait()

    H1 = 3 * NTERM
    H2 = CH * NTERM - H1

    def issue_gathers(s):
        pltpu.async_copy(table_hbm.at[idx_v.at[s, pl.ds(0, H1)]],
                         data_v.at[s, pl.ds(0, H1)], gsem)
        pltpu.async_copy(table_hbm.at[idx_v.at[s, pl.ds(H1, H2)]],
                         data_v.at[s, pl.ds(H1, H2)], gsem)

    def wait_gathers1(s):
        pltpu.make_async_copy(table_hbm.at[idx_v.at[s, pl.ds(0, H1)]],
                              data_v.at[s, pl.ds(0, H1)], gsem).wait()

    def wait_gathers2(s):
        pltpu.make_async_copy(table_hbm.at[idx_v.at[s, pl.ds(H1, H2)]],
                              data_v.at[s, pl.ds(H1, H2)], gsem).wait()

    def drain_out(s):
        pltpu.make_async_copy(out_v.at[s], out_hbm.at[pl.ds(0, CH * N_HEADS)],
                              osem).wait()

    pltpu.sync_copy(idx_hbm.at[pl.ds(base * NTERM, CH * NTERM)], idx_v.at[0])
    pltpu.sync_copy(w_hbm.at[pl.ds(base, CH)], w_v.at[0])
    issue_gathers(0)
    issue_idx(1, 1)
    issue_w(1, 1)

    def chunk_body(ci, carry):
        s = lax.rem(ci, 2)
        g0 = base + ci * CH

        @pl.when(ci + 1 < NCHUNK)
        def _():
            wait_io(1 - s)
            issue_gathers(1 - s)

        wait_gathers1(s)

        @pl.when(ci + 2 < NCHUNK)
        def _():
            issue_idx(ci + 2, s)

        @pl.when(ci >= 2)
        def _():
            drain_out(s)

        def g_h_body(gh, carry2):
            g = gh // N_HEADS
            h = gh % N_HEADS
            acc0 = jnp.zeros((16,), jnp.float32)
            acc1 = jnp.zeros((16,), jnp.float32)
            for corner in range(4):
                w16 = w_v[s, g, pl.ds(corner * NCOL + h * 16, 16)]
                for k in range(16):
                    wv_s = w16[k]
                    row = g * NTERM + corner * NCOL + h * 16 + k
                    d0, d1 = plsc.unpack(data_v[s, row, :],
                                         format=plsc.PackFormat.INTERLEAVED)
                    acc0 = acc0 + wv_s * d0
                    acc1 = acc1 + wv_s * d1
            out_v[s, gh, pl.ds(0, 16)] = acc0
            out_v[s, gh, pl.ds(16, 16)] = acc1
            return carry2

        lax.fori_loop(0, 3 * N_HEADS, g_h_body, 0)
        wait_gathers2(s)
        lax.fori_loop(3 * N_HEADS, CH * N_HEADS, g_h_body, 0)
        pltpu.async_copy(out_v.at[s],
                         out_hbm.at[pl.ds(g0 * N_HEADS, CH * N_HEADS)], osem)

        @pl.when(ci + 2 < NCHUNK)
        def _():
            issue_w(ci + 2, s)

        return carry

    lax.fori_loop(0, NCHUNK, chunk_body, 0)
    drain_out(0)
    drain_out(1)


def _sc_gather(table, idx_flat, w2):
    mesh = plsc.VectorSubcoreMesh(core_axis_name="c", subcore_axis_name="s")
    k = pl.kernel(
        _sc_body,
        out_type=jax.ShapeDtypeStruct((GROUPS * N_HEADS, D_PH), jnp.float32),
        mesh=mesh,
        scratch_types=[
            pltpu.VMEM((2, CH * NTERM), jnp.int32),
            pltpu.VMEM((2, CH, NTERM), jnp.float32),
            pltpu.VMEM((2, CH * NTERM, D_PH), jnp.bfloat16),
            pltpu.VMEM((2, CH * N_HEADS, D_PH), jnp.float32),
            pltpu.SemaphoreType.DMA,
            pltpu.SemaphoreType.DMA,
            pltpu.SemaphoreType.DMA,
        ],
        compiler_params=pltpu.CompilerParams(use_tc_tiling_on_sc=False,
                                             needs_layout_passes=False),
    )
    return k(table, idx_flat, w2)


def _proj_body(x_ref, w_ref, b_ref, o_ref):
    o_ref[...] = (jnp.dot(x_ref[...], w_ref[...],
                          preferred_element_type=jnp.float32) + b_ref[...])


def _out_proj(x, wout, bout2):
    nb = (B * LQ) // LB
    return pl.pallas_call(
        _proj_body,
        grid=(nb,),
        in_specs=[
            pl.BlockSpec((LB, D_MODEL), lambda i: (i, 0)),
            pl.BlockSpec((D_MODEL, D_MODEL), lambda i: (0, 0)),
            pl.BlockSpec((1, D_MODEL), lambda i: (0, 0)),
        ],
        out_specs=pl.BlockSpec((LB, D_MODEL), lambda i: (i, 0)),
        out_shape=jax.ShapeDtypeStruct((B * LQ, D_MODEL), jnp.float32),
    )(x, wout, bout2)


def kernel(query, reference_points, input_flatten, input_spatial_shapes,
           input_level_start_index, Wv, bv, Woff, boff, Wa, ba, Wout, bout):
    del input_spatial_shapes, input_level_start_index
    rp8 = reference_points.reshape(B, LQ, 8)
    woff_p = Woff.reshape(D_MODEL, NCOL, 2).transpose(0, 2, 1).reshape(D_MODEL, 2 * NCOL)
    boff_p = boff.reshape(NCOL, 2).transpose(1, 0).reshape(1, 2 * NCOL)

    value, idx, w = _prep(query, input_flatten, rp8, Wv, bv.reshape(1, -1),
                          woff_p, boff_p, Wa, ba.reshape(1, -1))

    table = value.reshape(B * LIN * N_HEADS, D_PH)
    idx_flat = idx.reshape(GROUPS * NTERM)
    w2 = w.reshape(GROUPS, NTERM)
    sampled = _sc_gather(table, idx_flat, w2)

    ar = jnp.arange(D_MODEL)
    perm = (ar // D_PH) * D_PH + jnp.asarray(_CH_PERM)[ar % D_PH]
    out = _out_proj(sampled.reshape(B * LQ, D_MODEL), Wout[perm], bout.reshape(1, -1))
    return out.reshape(B, LQ, D_MODEL)
